# Initial kernel scaffold; baseline (speedup 1.0000x reference)
import jax
import jax.numpy as jnp
from jax.experimental import pallas as pl


def kernel(x, edge_index, batch, W1, b1, g1, be1, W2, b2, g2, be2, W3, b3, g3, be3, fcW, fcb):
    raise NotImplementedError("write your pallas kernel here")


# speedup vs baseline: 13.3138x; 13.3138x over previous
import functools

import jax
import jax.numpy as jnp
from jax import lax
from jax.experimental import pallas as pl
from jax.experimental.pallas import tpu as pltpu
from jax.experimental.pallas import tpu_sc as plsc

N = 10000
E = 320000
NUM_GRAPHS = 64

NC = 2
NS = 16
K = 125
EROWS = E // K
IB = 16
CH = 200
NCHUNKS = N // CH
KITER = -(-NCHUNKS // NS)
R = 2000
GRID = N // R


def _mesh():
  return plsc.VectorSubcoreMesh(core_axis_name="c", subcore_axis_name="s")


def _sc_deg(dst2d, ones_rows, zero_rows):
  rows_per_worker = EROWS // (NC * NS)

  @functools.partial(
      pl.kernel,
      out_type=jax.ShapeDtypeStruct((NC, N, 128), jnp.float32),
      mesh=_mesh(),
      scratch_types=[
          pltpu.VMEM((IB, K), jnp.int32),
          pltpu.VMEM((K, 128), jnp.float32),
          pltpu.VMEM((CH, 128), jnp.float32),
          pltpu.VMEM_SHARED((N, 128), jnp.float32),
      ],
  )
  def deg_kernel(dst_hbm, ones_hbm, zeros_hbm, out_hbm, dst_c, ones, cbuf,
                 sh_deg):
    c = lax.axis_index("c")
    s = lax.axis_index("s")
    w = c * NS + s
    pltpu.sync_copy(ones_hbm, ones)
    pltpu.sync_copy(zeros_hbm, cbuf)

    def zchunk(k, _):
      idx = k * NS + s
      @pl.when(idx < NCHUNKS)
      def _():
        pltpu.sync_copy(cbuf, sh_deg.at[pl.ds(idx * CH, CH)])
      return 0
    lax.fori_loop(0, KITER, zchunk, 0)

    plsc.subcore_barrier()

    def chunk(ci, _):
      pltpu.sync_copy(dst_hbm.at[pl.ds(w * rows_per_worker + ci * IB, IB)],
                      dst_c)
      def scat(j, _):
        pltpu.sync_copy(ones, sh_deg.at[dst_c.at[j]], add=True)
        return 0
      lax.fori_loop(0, IB, scat, 0)
      return 0
    lax.fori_loop(0, rows_per_worker // IB, chunk, 0)

    plsc.subcore_barrier()

    def wchunk(k, _):
      idx = k * NS + s
      @pl.when(idx < NCHUNKS)
      def _():
        pltpu.sync_copy(sh_deg.at[pl.ds(idx * CH, CH)], cbuf)
        pltpu.sync_copy(cbuf, out_hbm.at[c, pl.ds(idx * CH, CH)])
      return 0
    lax.fori_loop(0, KITER, wchunk, 0)

  return deg_kernel(dst2d, ones_rows, zero_rows)


def _sc_agg_split(hs_lo, hs_hi, src2d, dst2d, zero_rows):
  rows_per_sub = EROWS // NS

  @functools.partial(
      pl.kernel,
      out_type=[jax.ShapeDtypeStruct((N, 128), jnp.float32),
                jax.ShapeDtypeStruct((N, 128), jnp.float32)],
      mesh=_mesh(),
      scratch_types=[
          pltpu.VMEM((IB, K), jnp.int32),
          pltpu.VMEM((IB, K), jnp.int32),
          pltpu.VMEM((K, 128), jnp.float32),
          pltpu.VMEM((CH, 128), jnp.float32),
          pltpu.VMEM_SHARED((N, 128), jnp.float32),
      ],
  )
  def agg_kernel(lo_hbm, hi_hbm, src_hbm, dst_hbm, zeros_hbm, out_lo, out_hi,
                 src_c, dst_c, buf, cbuf, sh_agg):
    c = lax.axis_index("c")
    s = lax.axis_index("s")
    pltpu.sync_copy(zeros_hbm, cbuf)

    def zchunk(k, _):
      idx = k * NS + s
      @pl.when(idx < NCHUNKS)
      def _():
        pltpu.sync_copy(cbuf, sh_agg.at[pl.ds(idx * CH, CH)])
      return 0
    lax.fori_loop(0, KITER, zchunk, 0)

    plsc.subcore_barrier()

    def run(tbl, out):
      def chunk(ci, _):
        base = s * rows_per_sub + ci * IB
        pltpu.sync_copy(src_hbm.at[pl.ds(base, IB)], src_c)
        pltpu.sync_copy(dst_hbm.at[pl.ds(base, IB)], dst_c)
        def edge_step(j, _):
          pltpu.sync_copy(tbl.at[src_c.at[j]], buf)
          pltpu.sync_copy(buf, sh_agg.at[dst_c.at[j]], add=True)
          return 0
        lax.fori_loop(0, IB, edge_step, 0)
        return 0
      lax.fori_loop(0, rows_per_sub // IB, chunk, 0)

      plsc.subcore_barrier()

      def out_chunk(k, _):
        idx = k * NS + s
        @pl.when(idx < NCHUNKS)
        def _():
          pltpu.sync_copy(sh_agg.at[pl.ds(idx * CH, CH)], cbuf)
          pltpu.sync_copy(cbuf, out.at[pl.ds(idx * CH, CH)])
        return 0
      lax.fori_loop(0, KITER, out_chunk, 0)

    @pl.when(c == 0)
    def _():
      run(lo_hbm, out_lo)

    @pl.when(c == 1)
    def _():
      run(hi_hbm, out_hi)

  return agg_kernel(hs_lo, hs_hi, src2d, dst2d, zero_rows)


def _sc_agg_full(hs, src2d, dst2d, zero_rows):
  rows_per_worker = EROWS // (NC * NS)

  @functools.partial(
      pl.kernel,
      out_type=jax.ShapeDtypeStruct((NC, N, 128), jnp.float32),
      mesh=_mesh(),
      scratch_types=[
          pltpu.VMEM((IB, K), jnp.int32),
          pltpu.VMEM((IB, K), jnp.int32),
          pltpu.VMEM((K, 128), jnp.float32),
          pltpu.VMEM((CH, 128), jnp.float32),
          pltpu.VMEM_SHARED((N, 128), jnp.float32),
      ],
  )
  def agg_kernel(hs_hbm, src_hbm, dst_hbm, zeros_hbm, out_hbm,
                 src_c, dst_c, buf, cbuf, sh_agg):
    c = lax.axis_index("c")
    s = lax.axis_index("s")
    w = c * NS + s
    pltpu.sync_copy(zeros_hbm, cbuf)

    def zchunk(k, _):
      idx = k * NS + s
      @pl.when(idx < NCHUNKS)
      def _():
        pltpu.sync_copy(cbuf, sh_agg.at[pl.ds(idx * CH, CH)])
      return 0
    lax.fori_loop(0, KITER, zchunk, 0)

    plsc.subcore_barrier()

    def chunk(ci, _):
      base = w * rows_per_worker + ci * IB
      pltpu.sync_copy(src_hbm.at[pl.ds(base, IB)], src_c)
      pltpu.sync_copy(dst_hbm.at[pl.ds(base, IB)], dst_c)
      def edge_step(j, _):
        pltpu.sync_copy(hs_hbm.at[src_c.at[j]], buf)
        pltpu.sync_copy(buf, sh_agg.at[dst_c.at[j]], add=True)
        return 0
      lax.fori_loop(0, IB, edge_step, 0)
      return 0
    lax.fori_loop(0, rows_per_worker // IB, chunk, 0)

    plsc.subcore_barrier()

    def wchunk(k, _):
      idx = k * NS + s
      @pl.when(idx < NCHUNKS)
      def _():
        pltpu.sync_copy(sh_agg.at[pl.ds(idx * CH, CH)], cbuf)
        pltpu.sync_copy(cbuf, out_hbm.at[c, pl.ds(idx * CH, CH)])
      return 0
    lax.fori_loop(0, KITER, wchunk, 0)

  return agg_kernel(hs, src2d, dst2d, zero_rows)


def _dinv_block(degp):
  d = degp[0, :, 0:1] + degp[1, :, 0:1] + 1.0
  return lax.rsqrt(d)


def _mm(a, b):
  return jnp.dot(a, b, preferred_element_type=jnp.float32,
                 precision=lax.Precision.HIGHEST)


def _tc_h(z, scale, shift, W, degp, din, dout, split, want_x=False):
  H = dout // 2

  def body(*refs):
    if scale is None:
      z_ref, w_ref, degp_ref = refs[:3]
      outs = refs[3:]
      xv = z_ref[...]
    else:
      z_ref, sc_ref, sh_ref, w_ref, degp_ref = refs[:5]
      outs = refs[5:]
      xv = jnp.maximum(z_ref[...] * sc_ref[...] + sh_ref[...], 0.0)
    dinv = _dinv_block(degp_ref)
    hs = _mm(xv, w_ref[...]) * dinv
    if split:
      outs[0][...] = hs[:, :H]
      outs[1][...] = hs[:, H:]
    else:
      outs[0][...] = hs
    if want_x:
      outs[-1][...] = xv

  in_specs = [pl.BlockSpec((R, din), lambda i: (i, 0))]
  ins = [z]
  if scale is not None:
    in_specs += [pl.BlockSpec((1, din), lambda i: (0, 0)),
                 pl.BlockSpec((1, din), lambda i: (0, 0))]
    ins += [scale, shift]
  in_specs += [pl.BlockSpec((din, dout), lambda i: (0, 0)),
               pl.BlockSpec((NC, R, 128), lambda i: (0, i, 0))]
  ins += [W, degp]

  if split:
    out_specs = [pl.BlockSpec((R, H), lambda i: (i, 0)),
                 pl.BlockSpec((R, H), lambda i: (i, 0))]
    out_shape = [jax.ShapeDtypeStruct((N, H), jnp.float32)] * 2
  else:
    out_specs = [pl.BlockSpec((R, dout), lambda i: (i, 0))]
    out_shape = [jax.ShapeDtypeStruct((N, dout), jnp.float32)]
  if want_x:
    out_specs.append(pl.BlockSpec((R, din), lambda i: (i, 0)))
    out_shape.append(jax.ShapeDtypeStruct((N, din), jnp.float32))

  return pl.pallas_call(
      body, grid=(GRID,), in_specs=in_specs, out_specs=out_specs,
      out_shape=out_shape)(*ins)


def _tc_z1(partials, hs, degp, b2d):
  def body(p_ref, hs_ref, degp_ref, b_ref, z_ref, st_ref):
    i = pl.program_id(0)
    dinv = _dinv_block(degp_ref)
    agg = p_ref[0] + p_ref[1] + hs_ref[...]
    z = agg * dinv + b_ref[...]
    z_ref[...] = z

    @pl.when(i == 0)
    def _():
      st_ref[...] = jnp.zeros_like(st_ref)

    st_ref[0:1, :] += jnp.sum(z, axis=0, keepdims=True)
    st_ref[1:2, :] += jnp.sum(z * z, axis=0, keepdims=True)

  return pl.pallas_call(
      body,
      grid=(GRID,),
      in_specs=[
          pl.BlockSpec((NC, R, 128), lambda i: (0, i, 0)),
          pl.BlockSpec((R, 128), lambda i: (i, 0)),
          pl.BlockSpec((NC, R, 128), lambda i: (0, i, 0)),
          pl.BlockSpec((1, 128), lambda i: (0, 0)),
      ],
      out_specs=[pl.BlockSpec((R, 128), lambda i: (i, 0)),
                 pl.BlockSpec((8, 128), lambda i: (0, 0))],
      out_shape=[jax.ShapeDtypeStruct((N, 128), jnp.float32),
                 jax.ShapeDtypeStruct((8, 128), jnp.float32)],
  )(partials, hs, degp, b2d)


def _tc_z2(a_lo, a_hi, hs_lo, hs_hi, degp, b2d):
  D = 256

  def body(alo_ref, ahi_ref, hlo_ref, hhi_ref, degp_ref, b_ref, z_ref, st_ref):
    i = pl.program_id(0)
    dinv = _dinv_block(degp_ref)
    agg = jnp.concatenate([alo_ref[...] + hlo_ref[...],
                           ahi_ref[...] + hhi_ref[...]], axis=1)
    z = agg * dinv + b_ref[...]
    z_ref[...] = z

    @pl.when(i == 0)
    def _():
      st_ref[...] = jnp.zeros_like(st_ref)

    st_ref[0:1, :] += jnp.sum(z, axis=0, keepdims=True)
    st_ref[1:2, :] += jnp.sum(z * z, axis=0, keepdims=True)

  half_spec = pl.BlockSpec((R, 128), lambda i: (i, 0))
  return pl.pallas_call(
      body,
      grid=(GRID,),
      in_specs=[
          half_spec, half_spec, half_spec, half_spec,
          pl.BlockSpec((NC, R, 128), lambda i: (0, i, 0)),
          pl.BlockSpec((1, D), lambda i: (0, 0)),
      ],
      out_specs=[pl.BlockSpec((R, D), lambda i: (i, 0)),
                 pl.BlockSpec((8, D), lambda i: (0, 0))],
      out_shape=[jax.ShapeDtypeStruct((N, D), jnp.float32),
                 jax.ShapeDtypeStruct((8, D), jnp.float32)],
  )(a_lo, a_hi, hs_lo, hs_hi, degp, b2d)


def _tc_pool(z3, scale, shift, x2, batch3d, fcW, fcb2d, D, F):
  def body(z_ref, sc_ref, sh_ref, x2_ref, b_ref, w_ref, fb_ref, out_ref,
           acc_ref, cnt_ref):
    i = pl.program_id(0)

    @pl.when(i == 0)
    def _():
      acc_ref[...] = jnp.zeros_like(acc_ref)
      cnt_ref[...] = jnp.zeros_like(cnt_ref)

    x3 = jnp.maximum(z_ref[...] * sc_ref[...] + sh_ref[...], 0.0) + x2_ref[...]
    b = b_ref[0, 0, :]
    gids = lax.broadcasted_iota(jnp.int32, (NUM_GRAPHS, R), 0)
    onehot = (b[None, :] == gids).astype(jnp.float32)
    acc_ref[...] += _mm(onehot, x3)
    csum = jnp.sum(onehot, axis=1, keepdims=True)
    cnt_ref[...] += jnp.broadcast_to(csum, (NUM_GRAPHS, 128))

    @pl.when(i == GRID - 1)
    def _():
      pooled = acc_ref[...] / jnp.maximum(cnt_ref[:, 0:1], 1.0)
      out_ref[...] = _mm(pooled, w_ref[...]) + fb_ref[...]

  return pl.pallas_call(
      body,
      grid=(GRID,),
      in_specs=[
          pl.BlockSpec((R, D), lambda i: (i, 0)),
          pl.BlockSpec((1, D), lambda i: (0, 0)),
          pl.BlockSpec((1, D), lambda i: (0, 0)),
          pl.BlockSpec((R, D), lambda i: (i, 0)),
          pl.BlockSpec((1, 1, R), lambda i: (i, 0, 0)),
          pl.BlockSpec((D, F), lambda i: (0, 0)),
          pl.BlockSpec((1, F), lambda i: (0, 0)),
      ],
      out_specs=pl.BlockSpec((NUM_GRAPHS, F), lambda i: (0, 0)),
      out_shape=jax.ShapeDtypeStruct((NUM_GRAPHS, F), jnp.float32),
      scratch_shapes=[pltpu.VMEM((NUM_GRAPHS, F), jnp.float32),
                      pltpu.VMEM((NUM_GRAPHS, 128), jnp.float32)],
  )(z3, scale, shift, x2, batch3d, fcW, fcb2d)


def _bn_coeffs(stats, g, be, eps=1e-5):
  mu = stats[0] / N
  var = stats[1] / N - mu * mu
  rstd = lax.rsqrt(var + eps)
  scale = g * rstd
  shift = be - mu * scale
  return scale[None, :], shift[None, :]


def kernel(x, edge_index, batch, W1, b1, g1, be1, W2, b2, g2, be2,
           W3, b3, g3, be3, fcW, fcb):
  src2d = edge_index[0].reshape(EROWS, K)
  dst2d = edge_index[1].reshape(EROWS, K)
  batch3d = batch.reshape(GRID, 1, R)
  ones_rows = jnp.ones((K, 128), jnp.float32)
  zero_rows = jnp.zeros((CH, 128), jnp.float32)

  degp = _sc_deg(dst2d, ones_rows, zero_rows)

  (hs1,) = _tc_h(x, None, None, W1, degp, 128, 128, split=False)
  p1 = _sc_agg_full(hs1, src2d, dst2d, zero_rows)
  z1, st1 = _tc_z1(p1, hs1, degp, b1[None, :])
  sc1, sh1 = _bn_coeffs(st1, g1, be1)

  hs_lo, hs_hi = _tc_h(z1, sc1, sh1, W2, degp, 128, 256, split=True)
  a_lo, a_hi = _sc_agg_split(hs_lo, hs_hi, src2d, dst2d, zero_rows)
  z2, st2 = _tc_z2(a_lo, a_hi, hs_lo, hs_hi, degp, b2[None, :])
  sc2, sh2 = _bn_coeffs(st2, g2, be2)

  hs_lo, hs_hi, x2 = _tc_h(z2, sc2, sh2, W3, degp, 256, 256, split=True,
                           want_x=True)
  a_lo, a_hi = _sc_agg_split(hs_lo, hs_hi, src2d, dst2d, zero_rows)
  z3, st3 = _tc_z2(a_lo, a_hi, hs_lo, hs_hi, degp, b3[None, :])
  sc3, sh3 = _bn_coeffs(st3, g3, be3)

  return _tc_pool(z3, sc3, sh3, x2, batch3d, fcW, fcb[None, :], 256, 256)


# speedup vs baseline: 19.6245x; 1.4740x over previous
import functools

import jax
import jax.numpy as jnp
from jax import lax
from jax.experimental import pallas as pl
from jax.experimental.pallas import tpu as pltpu
from jax.experimental.pallas import tpu_sc as plsc

N = 10000
E = 320000
NUM_GRAPHS = 64

NC = 2
NS = 16
K = 125
EROWS = E // K
CH = 200
NCHUNKS = N // CH
KITER = -(-NCHUNKS // NS)
R = 2000
GRID = N // R


def _mesh():
  return plsc.VectorSubcoreMesh(core_axis_name="c", subcore_axis_name="s")


def _rr_copy(src, dst, s):
  def go(k, _):
    idx = k * NS + s
    @pl.when(idx < NCHUNKS)
    def _():
      pltpu.sync_copy(src.at[pl.ds(idx * CH, CH)], dst.at[pl.ds(idx * CH, CH)])
    return 0
  lax.fori_loop(0, KITER, go, 0)


def _edge_pipeline(tbl, sh_agg, src_c, dst_c, bufs, gsems, ssems, nrows):
  gd = [None] * nrows
  sd = [None] * nrows
  gd[0] = pltpu.async_copy(tbl.at[src_c.at[0]], bufs[0], gsems[0])
  for j in range(1, nrows):
    p = j % 2
    if j >= 2:
      sd[j - 2].wait()
    gd[j] = pltpu.async_copy(tbl.at[src_c.at[j]], bufs[p], gsems[p])
    gd[j - 1].wait()
    q = (j - 1) % 2
    sd[j - 1] = pltpu.async_copy(bufs[q], sh_agg.at[dst_c.at[j - 1]],
                                 ssems[q], add=True)
  gd[nrows - 1].wait()
  q = (nrows - 1) % 2
  sd[nrows - 1] = pltpu.async_copy(bufs[q], sh_agg.at[dst_c.at[nrows - 1]],
                                   ssems[q], add=True)
  sd[nrows - 2].wait()
  sd[nrows - 1].wait()


def _sc_deg(dst2d, ones_rows, zeros_full):
  rows_per_worker = EROWS // (NC * NS)
  FIRE = 8

  @functools.partial(
      pl.kernel,
      out_type=jax.ShapeDtypeStruct((NC, N, 128), jnp.float32),
      mesh=_mesh(),
      scratch_types=[
          pltpu.VMEM((rows_per_worker, K), jnp.int32),
          pltpu.VMEM((K, 128), jnp.float32),
          pltpu.VMEM_SHARED((N, 128), jnp.float32),
          pltpu.SemaphoreType.DMA,
      ],
  )
  def deg_kernel(dst_hbm, ones_hbm, zeros_hbm, out_hbm, dst_v, ones, sh_deg,
                 sem):
    c = lax.axis_index("c")
    s = lax.axis_index("s")
    w = c * NS + s
    pltpu.sync_copy(ones_hbm, ones)
    pltpu.sync_copy(dst_hbm.at[pl.ds(w * rows_per_worker, rows_per_worker)],
                    dst_v)
    _rr_copy(zeros_hbm, sh_deg, s)

    plsc.subcore_barrier()

    def grp(gi, _):
      ds = []
      for j in range(FIRE):
        ds.append(pltpu.async_copy(
            ones, sh_deg.at[dst_v.at[gi * FIRE + j]], sem, add=True))
      for d in ds:
        d.wait()
      return 0
    lax.fori_loop(0, rows_per_worker // FIRE, grp, 0)

    plsc.subcore_barrier()
    _rr_copy(sh_deg, out_hbm.at[c], s)

  return deg_kernel(dst2d, ones_rows, zeros_full)


def _sc_agg_split(hs_lo, hs_hi, src2d, dst2d, zeros_full):
  rows_per_sub = EROWS // NS
  IBS = 32
  n_chunks = rows_per_sub // IBS

  @functools.partial(
      pl.kernel,
      out_type=[jax.ShapeDtypeStruct((N, 128), jnp.float32),
                jax.ShapeDtypeStruct((N, 128), jnp.float32)],
      mesh=_mesh(),
      scratch_types=[
          pltpu.VMEM((IBS, K), jnp.int32),
          pltpu.VMEM((IBS, K), jnp.int32),
          pltpu.VMEM((K, 128), jnp.float32),
          pltpu.VMEM((K, 128), jnp.float32),
          pltpu.VMEM_SHARED((N, 128), jnp.float32),
          pltpu.SemaphoreType.DMA,
          pltpu.SemaphoreType.DMA,
          pltpu.SemaphoreType.DMA,
          pltpu.SemaphoreType.DMA,
      ],
  )
  def agg_kernel(lo_hbm, hi_hbm, src_hbm, dst_hbm, zeros_hbm, out_lo, out_hi,
                 src_c, dst_c, b0, b1, sh_agg, gs0, gs1, ss0, ss1):
    c = lax.axis_index("c")
    s = lax.axis_index("s")
    _rr_copy(zeros_hbm, sh_agg, s)
    plsc.subcore_barrier()

    def run(tbl, out):
      def chunk(ci, _):
        base = s * rows_per_sub + ci * IBS
        pltpu.sync_copy(src_hbm.at[pl.ds(base, IBS)], src_c)
        pltpu.sync_copy(dst_hbm.at[pl.ds(base, IBS)], dst_c)
        _edge_pipeline(tbl, sh_agg, src_c, dst_c, (b0, b1), (gs0, gs1),
                       (ss0, ss1), IBS)
        return 0
      lax.fori_loop(0, n_chunks, chunk, 0)

      plsc.subcore_barrier()
      _rr_copy(sh_agg, out, s)

    @pl.when(c == 0)
    def _():
      run(lo_hbm, out_lo)

    @pl.when(c == 1)
    def _():
      run(hi_hbm, out_hi)

  return agg_kernel(hs_lo, hs_hi, src2d, dst2d, zeros_full)


def _sc_agg_full(hs, src2d, dst2d, zeros_full):
  rows_per_worker = EROWS // (NC * NS)
  IBS = 16
  n_chunks = rows_per_worker // IBS

  @functools.partial(
      pl.kernel,
      out_type=jax.ShapeDtypeStruct((NC, N, 128), jnp.float32),
      mesh=_mesh(),
      scratch_types=[
          pltpu.VMEM((IBS, K), jnp.int32),
          pltpu.VMEM((IBS, K), jnp.int32),
          pltpu.VMEM((K, 128), jnp.float32),
          pltpu.VMEM((K, 128), jnp.float32),
          pltpu.VMEM_SHARED((N, 128), jnp.float32),
          pltpu.SemaphoreType.DMA,
          pltpu.SemaphoreType.DMA,
          pltpu.SemaphoreType.DMA,
          pltpu.SemaphoreType.DMA,
      ],
  )
  def agg_kernel(hs_hbm, src_hbm, dst_hbm, zeros_hbm, out_hbm,
                 src_c, dst_c, b0, b1, sh_agg, gs0, gs1, ss0, ss1):
    c = lax.axis_index("c")
    s = lax.axis_index("s")
    w = c * NS + s
    _rr_copy(zeros_hbm, sh_agg, s)
    plsc.subcore_barrier()

    def chunk(ci, _):
      base = w * rows_per_worker + ci * IBS
      pltpu.sync_copy(src_hbm.at[pl.ds(base, IBS)], src_c)
      pltpu.sync_copy(dst_hbm.at[pl.ds(base, IBS)], dst_c)
      _edge_pipeline(hs_hbm, sh_agg, src_c, dst_c, (b0, b1), (gs0, gs1),
                     (ss0, ss1), IBS)
      return 0
    lax.fori_loop(0, n_chunks, chunk, 0)

    plsc.subcore_barrier()
    _rr_copy(sh_agg, out_hbm.at[c], s)

  return agg_kernel(hs, src2d, dst2d, zeros_full)


def _dinv_block(degp):
  d = degp[0, :, 0:1] + degp[1, :, 0:1] + 1.0
  return lax.rsqrt(d)


def _mm(a, b):
  return jnp.dot(a, b, preferred_element_type=jnp.float32,
                 precision=lax.Precision.HIGHEST)


def _tc_h(z, scale, shift, W, degp, din, dout, split, want_x=False):
  H = dout // 2

  def body(*refs):
    if scale is None:
      z_ref, w_ref, degp_ref = refs[:3]
      outs = refs[3:]
      xv = z_ref[...]
    else:
      z_ref, sc_ref, sh_ref, w_ref, degp_ref = refs[:5]
      outs = refs[5:]
      xv = jnp.maximum(z_ref[...] * sc_ref[...] + sh_ref[...], 0.0)
    dinv = _dinv_block(degp_ref)
    hs = _mm(xv, w_ref[...]) * dinv
    if split:
      outs[0][...] = hs[:, :H]
      outs[1][...] = hs[:, H:]
    else:
      outs[0][...] = hs
    if want_x:
      outs[-1][...] = xv

  in_specs = [pl.BlockSpec((R, din), lambda i: (i, 0))]
  ins = [z]
  if scale is not None:
    in_specs += [pl.BlockSpec((1, din), lambda i: (0, 0)),
                 pl.BlockSpec((1, din), lambda i: (0, 0))]
    ins += [scale, shift]
  in_specs += [pl.BlockSpec((din, dout), lambda i: (0, 0)),
               pl.BlockSpec((NC, R, 128), lambda i: (0, i, 0))]
  ins += [W, degp]

  if split:
    out_specs = [pl.BlockSpec((R, H), lambda i: (i, 0)),
                 pl.BlockSpec((R, H), lambda i: (i, 0))]
    out_shape = [jax.ShapeDtypeStruct((N, H), jnp.float32)] * 2
  else:
    out_specs = [pl.BlockSpec((R, dout), lambda i: (i, 0))]
    out_shape = [jax.ShapeDtypeStruct((N, dout), jnp.float32)]
  if want_x:
    out_specs.append(pl.BlockSpec((R, din), lambda i: (i, 0)))
    out_shape.append(jax.ShapeDtypeStruct((N, din), jnp.float32))

  return pl.pallas_call(
      body, grid=(GRID,), in_specs=in_specs, out_specs=out_specs,
      out_shape=out_shape)(*ins)


def _tc_z1(partials, hs, degp, b2d):
  def body(p_ref, hs_ref, degp_ref, b_ref, z_ref, st_ref):
    i = pl.program_id(0)
    dinv = _dinv_block(degp_ref)
    agg = p_ref[0] + p_ref[1] + hs_ref[...]
    z = agg * dinv + b_ref[...]
    z_ref[...] = z

    @pl.when(i == 0)
    def _():
      st_ref[...] = jnp.zeros_like(st_ref)

    st_ref[0:1, :] += jnp.sum(z, axis=0, keepdims=True)
    st_ref[1:2, :] += jnp.sum(z * z, axis=0, keepdims=True)

  return pl.pallas_call(
      body,
      grid=(GRID,),
      in_specs=[
          pl.BlockSpec((NC, R, 128), lambda i: (0, i, 0)),
          pl.BlockSpec((R, 128), lambda i: (i, 0)),
          pl.BlockSpec((NC, R, 128), lambda i: (0, i, 0)),
          pl.BlockSpec((1, 128), lambda i: (0, 0)),
      ],
      out_specs=[pl.BlockSpec((R, 128), lambda i: (i, 0)),
                 pl.BlockSpec((8, 128), lambda i: (0, 0))],
      out_shape=[jax.ShapeDtypeStruct((N, 128), jnp.float32),
                 jax.ShapeDtypeStruct((8, 128), jnp.float32)],
  )(partials, hs, degp, b2d)


def _tc_z2(a_lo, a_hi, hs_lo, hs_hi, degp, b2d):
  D = 256

  def body(alo_ref, ahi_ref, hlo_ref, hhi_ref, degp_ref, b_ref, z_ref, st_ref):
    i = pl.program_id(0)
    dinv = _dinv_block(degp_ref)
    agg = jnp.concatenate([alo_ref[...] + hlo_ref[...],
                           ahi_ref[...] + hhi_ref[...]], axis=1)
    z = agg * dinv + b_ref[...]
    z_ref[...] = z

    @pl.when(i == 0)
    def _():
      st_ref[...] = jnp.zeros_like(st_ref)

    st_ref[0:1, :] += jnp.sum(z, axis=0, keepdims=True)
    st_ref[1:2, :] += jnp.sum(z * z, axis=0, keepdims=True)

  half_spec = pl.BlockSpec((R, 128), lambda i: (i, 0))
  return pl.pallas_call(
      body,
      grid=(GRID,),
      in_specs=[
          half_spec, half_spec, half_spec, half_spec,
          pl.BlockSpec((NC, R, 128), lambda i: (0, i, 0)),
          pl.BlockSpec((1, D), lambda i: (0, 0)),
      ],
      out_specs=[pl.BlockSpec((R, D), lambda i: (i, 0)),
                 pl.BlockSpec((8, D), lambda i: (0, 0))],
      out_shape=[jax.ShapeDtypeStruct((N, D), jnp.float32),
                 jax.ShapeDtypeStruct((8, D), jnp.float32)],
  )(a_lo, a_hi, hs_lo, hs_hi, degp, b2d)


def _tc_pool(z3, scale, shift, x2, batch3d, fcW, fcb2d, D, F):
  def body(z_ref, sc_ref, sh_ref, x2_ref, b_ref, w_ref, fb_ref, out_ref,
           acc_ref, cnt_ref):
    i = pl.program_id(0)

    @pl.when(i == 0)
    def _():
      acc_ref[...] = jnp.zeros_like(acc_ref)
      cnt_ref[...] = jnp.zeros_like(cnt_ref)

    x3 = jnp.maximum(z_ref[...] * sc_ref[...] + sh_ref[...], 0.0) + x2_ref[...]
    b = b_ref[0, 0, :]
    gids = lax.broadcasted_iota(jnp.int32, (NUM_GRAPHS, R), 0)
    onehot = (b[None, :] == gids).astype(jnp.float32)
    acc_ref[...] += _mm(onehot, x3)
    csum = jnp.sum(onehot, axis=1, keepdims=True)
    cnt_ref[...] += jnp.broadcast_to(csum, (NUM_GRAPHS, 128))

    @pl.when(i == GRID - 1)
    def _():
      pooled = acc_ref[...] / jnp.maximum(cnt_ref[:, 0:1], 1.0)
      out_ref[...] = _mm(pooled, w_ref[...]) + fb_ref[...]

  return pl.pallas_call(
      body,
      grid=(GRID,),
      in_specs=[
          pl.BlockSpec((R, D), lambda i: (i, 0)),
          pl.BlockSpec((1, D), lambda i: (0, 0)),
          pl.BlockSpec((1, D), lambda i: (0, 0)),
          pl.BlockSpec((R, D), lambda i: (i, 0)),
          pl.BlockSpec((1, 1, R), lambda i: (i, 0, 0)),
          pl.BlockSpec((D, F), lambda i: (0, 0)),
          pl.BlockSpec((1, F), lambda i: (0, 0)),
      ],
      out_specs=pl.BlockSpec((NUM_GRAPHS, F), lambda i: (0, 0)),
      out_shape=jax.ShapeDtypeStruct((NUM_GRAPHS, F), jnp.float32),
      scratch_shapes=[pltpu.VMEM((NUM_GRAPHS, F), jnp.float32),
                      pltpu.VMEM((NUM_GRAPHS, 128), jnp.float32)],
  )(z3, scale, shift, x2, batch3d, fcW, fcb2d)


def _bn_coeffs(stats, g, be, eps=1e-5):
  mu = stats[0] / N
  var = stats[1] / N - mu * mu
  rstd = lax.rsqrt(var + eps)
  scale = g * rstd
  shift = be - mu * scale
  return scale[None, :], shift[None, :]


def kernel(x, edge_index, batch, W1, b1, g1, be1, W2, b2, g2, be2,
           W3, b3, g3, be3, fcW, fcb):
  src2d = edge_index[0].reshape(EROWS, K)
  dst2d = edge_index[1].reshape(EROWS, K)
  batch3d = batch.reshape(GRID, 1, R)
  ones_rows = jnp.ones((K, 128), jnp.float32)
  zeros_full = jnp.zeros((N, 128), jnp.float32)

  degp = _sc_deg(dst2d, ones_rows, zeros_full)

  (hs1,) = _tc_h(x, None, None, W1, degp, 128, 128, split=False)
  p1 = _sc_agg_full(hs1, src2d, dst2d, zeros_full)
  z1, st1 = _tc_z1(p1, hs1, degp, b1[None, :])
  sc1, sh1 = _bn_coeffs(st1, g1, be1)

  hs_lo, hs_hi = _tc_h(z1, sc1, sh1, W2, degp, 128, 256, split=True)
  a_lo, a_hi = _sc_agg_split(hs_lo, hs_hi, src2d, dst2d, zeros_full)
  z2, st2 = _tc_z2(a_lo, a_hi, hs_lo, hs_hi, degp, b2[None, :])
  sc2, sh2 = _bn_coeffs(st2, g2, be2)

  hs_lo, hs_hi, x2 = _tc_h(z2, sc2, sh2, W3, degp, 256, 256, split=True,
                           want_x=True)
  a_lo, a_hi = _sc_agg_split(hs_lo, hs_hi, src2d, dst2d, zeros_full)
  z3, st3 = _tc_z2(a_lo, a_hi, hs_lo, hs_hi, degp, b3[None, :])
  sc3, sh3 = _bn_coeffs(st3, g3, be3)

  return _tc_pool(z3, sc3, sh3, x2, batch3d, fcW, fcb[None, :], 256, 256)


# speedup vs baseline: 19.6455x; 1.0011x over previous
import functools

import jax
import jax.numpy as jnp
from jax import lax
from jax.experimental import pallas as pl
from jax.experimental.pallas import tpu as pltpu
from jax.experimental.pallas import tpu_sc as plsc

N = 10000
E = 320000
NUM_GRAPHS = 64

NC = 2
NS = 16
NW = NC * NS
K = 125
EROWS = E // K
CH = 200
NCHUNKS = N // CH
KITER = -(-NCHUNKS // NS)
NH = 10240
R = 2000
GRID = N // R
EPS = 1e-5


def _mesh():
  return plsc.VectorSubcoreMesh(core_axis_name="c", subcore_axis_name="s")


def _rr_copy(src, dst, s):
  def go(k, _):
    idx = k * NS + s
    @pl.when(idx < NCHUNKS)
    def _():
      pltpu.sync_copy(src.at[pl.ds(idx * CH, CH)], dst.at[pl.ds(idx * CH, CH)])
    return 0
  lax.fori_loop(0, KITER, go, 0)


def _edge_pipeline(tbl, sh_agg, src_c, dst_c, bufs, gsems, ssems, nrows):
  gd = [None] * nrows
  sd = [None] * nrows
  gd[0] = pltpu.async_copy(tbl.at[src_c.at[0]], bufs[0], gsems[0])
  for j in range(1, nrows):
    p = j % 2
    if j >= 2:
      sd[j - 2].wait()
    gd[j] = pltpu.async_copy(tbl.at[src_c.at[j]], bufs[p], gsems[p])
    gd[j - 1].wait()
    q = (j - 1) % 2
    sd[j - 1] = pltpu.async_copy(bufs[q], sh_agg.at[dst_c.at[j - 1]],
                                 ssems[q], add=True)
  gd[nrows - 1].wait()
  q = (nrows - 1) % 2
  sd[nrows - 1] = pltpu.async_copy(bufs[q], sh_agg.at[dst_c.at[nrows - 1]],
                                   ssems[q], add=True)
  sd[nrows - 2].wait()
  sd[nrows - 1].wait()


def _sc_deg(dst2d, ones_rows, zeros_full):
  rows_per_worker = EROWS // NW
  FIRE = 8

  @functools.partial(
      pl.kernel,
      out_type=jax.ShapeDtypeStruct((NC, N, 128), jnp.float32),
      mesh=_mesh(),
      scratch_types=[
          pltpu.VMEM((rows_per_worker, K), jnp.int32),
          pltpu.VMEM((K, 128), jnp.float32),
          pltpu.VMEM_SHARED((N, 128), jnp.float32),
          pltpu.SemaphoreType.DMA,
      ],
  )
  def deg_kernel(dst_hbm, ones_hbm, zeros_hbm, out_hbm, dst_v, ones, sh_deg,
                 sem):
    c = lax.axis_index("c")
    s = lax.axis_index("s")
    w = c * NS + s
    pltpu.sync_copy(ones_hbm, ones)
    pltpu.sync_copy(dst_hbm.at[pl.ds(w * rows_per_worker, rows_per_worker)],
                    dst_v)
    _rr_copy(zeros_hbm, sh_deg, s)

    plsc.subcore_barrier()

    def grp(gi, _):
      ds = []
      for j in range(FIRE):
        ds.append(pltpu.async_copy(
            ones, sh_deg.at[dst_v.at[gi * FIRE + j]], sem, add=True))
      for d in ds:
        d.wait()
      return 0
    lax.fori_loop(0, rows_per_worker // FIRE, grp, 0)

    plsc.subcore_barrier()
    _rr_copy(sh_deg, out_hbm.at[c], s)

  return deg_kernel(dst2d, ones_rows, zeros_full)


def _sc_agg_split(hs_lo, hs_hi, src2d, dst2d, zeros_full):
  rows_per_sub = EROWS // NS
  IBS = 32
  n_chunks = rows_per_sub // IBS

  @functools.partial(
      pl.kernel,
      out_type=[jax.ShapeDtypeStruct((N, 128), jnp.float32),
                jax.ShapeDtypeStruct((N, 128), jnp.float32)],
      mesh=_mesh(),
      scratch_types=[
          pltpu.VMEM((IBS, K), jnp.int32),
          pltpu.VMEM((IBS, K), jnp.int32),
          pltpu.VMEM((K, 128), jnp.float32),
          pltpu.VMEM((K, 128), jnp.float32),
          pltpu.VMEM_SHARED((N, 128), jnp.float32),
          pltpu.SemaphoreType.DMA,
          pltpu.SemaphoreType.DMA,
          pltpu.SemaphoreType.DMA,
          pltpu.SemaphoreType.DMA,
      ],
  )
  def agg_kernel(lo_hbm, hi_hbm, src_hbm, dst_hbm, zeros_hbm, out_lo, out_hi,
                 src_c, dst_c, b0, b1, sh_agg, gs0, gs1, ss0, ss1):
    c = lax.axis_index("c")
    s = lax.axis_index("s")
    _rr_copy(zeros_hbm, sh_agg, s)
    plsc.subcore_barrier()

    def run(tbl, out):
      def chunk(ci, _):
        base = s * rows_per_sub + ci * IBS
        pltpu.sync_copy(src_hbm.at[pl.ds(base, IBS)], src_c)
        pltpu.sync_copy(dst_hbm.at[pl.ds(base, IBS)], dst_c)
        _edge_pipeline(tbl, sh_agg, src_c, dst_c, (b0, b1), (gs0, gs1),
                       (ss0, ss1), IBS)
        return 0
      lax.fori_loop(0, n_chunks, chunk, 0)

      plsc.subcore_barrier()
      _rr_copy(sh_agg, out, s)

    @pl.when(c == 0)
    def _():
      run(lo_hbm, out_lo)

    @pl.when(c == 1)
    def _():
      run(hi_hbm, out_hi)

  return agg_kernel(hs_lo, hs_hi, src2d, dst2d, zeros_full)


def _sc_agg_full(hs, src2d, dst2d, zeros_full):
  rows_per_worker = EROWS // NW
  IBS = 16
  n_chunks = rows_per_worker // IBS

  @functools.partial(
      pl.kernel,
      out_type=jax.ShapeDtypeStruct((NC, N, 128), jnp.float32),
      mesh=_mesh(),
      scratch_types=[
          pltpu.VMEM((IBS, K), jnp.int32),
          pltpu.VMEM((IBS, K), jnp.int32),
          pltpu.VMEM((K, 128), jnp.float32),
          pltpu.VMEM((K, 128), jnp.float32),
          pltpu.VMEM_SHARED((N, 128), jnp.float32),
          pltpu.SemaphoreType.DMA,
          pltpu.SemaphoreType.DMA,
          pltpu.SemaphoreType.DMA,
          pltpu.SemaphoreType.DMA,
      ],
  )
  def agg_kernel(hs_hbm, src_hbm, dst_hbm, zeros_hbm, out_hbm,
                 src_c, dst_c, b0, b1, sh_agg, gs0, gs1, ss0, ss1):
    c = lax.axis_index("c")
    s = lax.axis_index("s")
    w = c * NS + s
    _rr_copy(zeros_hbm, sh_agg, s)
    plsc.subcore_barrier()

    def chunk(ci, _):
      base = w * rows_per_worker + ci * IBS
      pltpu.sync_copy(src_hbm.at[pl.ds(base, IBS)], src_c)
      pltpu.sync_copy(dst_hbm.at[pl.ds(base, IBS)], dst_c)
      _edge_pipeline(hs_hbm, sh_agg, src_c, dst_c, (b0, b1), (gs0, gs1),
                     (ss0, ss1), IBS)
      return 0
    lax.fori_loop(0, n_chunks, chunk, 0)

    plsc.subcore_barrier()
    _rr_copy(sh_agg, out_hbm.at[c], s)

  return agg_kernel(hs, src2d, dst2d, zeros_full)


def _mm(a, b):
  return jnp.dot(a, b, preferred_element_type=jnp.float32,
                 precision=lax.Precision.HIGHEST)


def _tc_first(x, W1, degp):
  def body(x_ref, w_ref, degp_ref, hs_ref, dinv_ref):
    d = degp_ref[0, :, 0:1] + degp_ref[1, :, 0:1] + 1.0
    dinv = lax.rsqrt(d)
    dinv_ref[...] = dinv
    hs_ref[...] = _mm(x_ref[...], w_ref[...]) * dinv

  return pl.pallas_call(
      body,
      grid=(GRID,),
      in_specs=[
          pl.BlockSpec((R, 128), lambda j: (j, 0)),
          pl.BlockSpec((128, 128), lambda j: (0, 0)),
          pl.BlockSpec((NC, R, 128), lambda j: (0, j, 0)),
      ],
      out_specs=[pl.BlockSpec((R, 128), lambda j: (j, 0)),
                 pl.BlockSpec((R, 1), lambda j: (j, 0))],
      out_shape=[jax.ShapeDtypeStruct((N, 128), jnp.float32),
                 jax.ShapeDtypeStruct((N, 1), jnp.float32)],
  )(x, W1, degp)


def _tc_layer(aggs, hss, dinv, bgbe, W, din, dout, want_x=False):
  H = dout // 2
  n_agg = len(aggs)
  n_hs = len(hss)

  def body(*refs):
    (agg_refs, hs_refs), rest = (refs[:n_agg], refs[n_agg:n_agg + n_hs]), \
        refs[n_agg + n_hs:]
    dinv_ref, b_ref, g_ref, be_ref, w_ref = rest[:5]
    outs = rest[5:-1]
    st_ref = rest[-1]
    p = pl.program_id(0)
    j = pl.program_id(1)

    if n_agg == 1:
      a3 = agg_refs[0][...]
      agg = a3[0] + a3[1]
    else:
      agg = jnp.concatenate([agg_refs[0][...], agg_refs[1][...]], axis=1)
    hs = hs_refs[0][...] if n_hs == 1 else \
        jnp.concatenate([hs_refs[0][...], hs_refs[1][...]], axis=1)
    z = (agg + hs) * dinv_ref[...] + b_ref[...]

    @pl.when(jnp.logical_and(p == 0, j == 0))
    def _():
      st_ref[...] = jnp.zeros_like(st_ref)

    @pl.when(p == 0)
    def _():
      st_ref[0:1, :] += jnp.sum(z, axis=0, keepdims=True)
      st_ref[1:2, :] += jnp.sum(z * z, axis=0, keepdims=True)

    @pl.when(p == 1)
    def _():
      @pl.when(j == 0)
      def _():
        mu = st_ref[0:1, :] / N
        var = st_ref[1:2, :] / N - mu * mu
        scale = g_ref[...] * lax.rsqrt(var + EPS)
        st_ref[2:3, :] = scale
        st_ref[3:4, :] = be_ref[...] - mu * scale

      xv = jnp.maximum(z * st_ref[2:3, :] + st_ref[3:4, :], 0.0)
      hs_next = _mm(xv, w_ref[...]) * dinv_ref[...]
      outs[0][...] = hs_next[:, :H]
      outs[1][...] = hs_next[:, H:]
      if want_x:
        outs[2][...] = xv

  in_specs = []
  ins = []
  for a in aggs:
    if a.ndim == 3:
      in_specs.append(pl.BlockSpec((NC, R, 128), lambda p, j: (0, j, 0)))
    else:
      in_specs.append(pl.BlockSpec((R, 128), lambda p, j: (j, 0)))
    ins.append(a)
  for h in hss:
    in_specs.append(pl.BlockSpec((R, h.shape[1]), lambda p, j: (j, 0)))
    ins.append(h)
  in_specs.append(pl.BlockSpec((R, 1), lambda p, j: (j, 0)))
  ins.append(dinv)
  for v in bgbe:
    in_specs.append(pl.BlockSpec((1, din), lambda p, j: (0, 0)))
    ins.append(v[None, :])
  in_specs.append(pl.BlockSpec((din, dout), lambda p, j: (0, 0)))
  ins.append(W)

  out_specs = [pl.BlockSpec((R, H), lambda p, j: (j, 0)),
               pl.BlockSpec((R, H), lambda p, j: (j, 0))]
  out_shape = [jax.ShapeDtypeStruct((N, H), jnp.float32)] * 2
  if want_x:
    out_specs.append(pl.BlockSpec((R, din), lambda p, j: (j, 0)))
    out_shape.append(jax.ShapeDtypeStruct((N, din), jnp.float32))

  return pl.pallas_call(
      body,
      grid=(2, GRID),
      in_specs=in_specs,
      out_specs=out_specs,
      out_shape=out_shape,
      scratch_shapes=[pltpu.VMEM((8, din), jnp.float32)],
  )(*ins)


def _tc_last(a_lo, a_hi, hs_lo, hs_hi, dinv, bgbe, x2, batch3d, fcW, fcb):
  D, F = 256, 256

  def body(alo_ref, ahi_ref, hlo_ref, hhi_ref, dinv_ref, b_ref, g_ref,
           be_ref, x2_ref, bt_ref, w_ref, fb_ref, out_ref, st_ref, acc_ref,
           cnt_ref):
    p = pl.program_id(0)
    j = pl.program_id(1)
    agg = jnp.concatenate([alo_ref[...], ahi_ref[...]], axis=1)
    hs = jnp.concatenate([hlo_ref[...], hhi_ref[...]], axis=1)
    z = (agg + hs) * dinv_ref[...] + b_ref[...]

    @pl.when(jnp.logical_and(p == 0, j == 0))
    def _():
      st_ref[...] = jnp.zeros_like(st_ref)
      acc_ref[...] = jnp.zeros_like(acc_ref)
      cnt_ref[...] = jnp.zeros_like(cnt_ref)

    @pl.when(p == 0)
    def _():
      st_ref[0:1, :] += jnp.sum(z, axis=0, keepdims=True)
      st_ref[1:2, :] += jnp.sum(z * z, axis=0, keepdims=True)

    @pl.when(p == 1)
    def _():
      @pl.when(j == 0)
      def _():
        mu = st_ref[0:1, :] / N
        var = st_ref[1:2, :] / N - mu * mu
        scale = g_ref[...] * lax.rsqrt(var + EPS)
        st_ref[2:3, :] = scale
        st_ref[3:4, :] = be_ref[...] - mu * scale

      x3 = jnp.maximum(z * st_ref[2:3, :] + st_ref[3:4, :], 0.0) + x2_ref[...]
      b = bt_ref[0, 0, :]
      gids = lax.broadcasted_iota(jnp.int32, (NUM_GRAPHS, R), 0)
      onehot = (b[None, :] == gids).astype(jnp.float32)
      acc_ref[...] += _mm(onehot, x3)
      csum = jnp.sum(onehot, axis=1, keepdims=True)
      cnt_ref[...] += jnp.broadcast_to(csum, (NUM_GRAPHS, 128))

      @pl.when(j == GRID - 1)
      def _():
        pooled = acc_ref[...] / jnp.maximum(cnt_ref[:, 0:1], 1.0)
        out_ref[...] = _mm(pooled, w_ref[...]) + fb_ref[...]

  half = pl.BlockSpec((R, 128), lambda p, j: (j, 0))
  vec = pl.BlockSpec((1, D), lambda p, j: (0, 0))
  b2, g2, be2 = (v[None, :] for v in bgbe)
  return pl.pallas_call(
      body,
      grid=(2, GRID),
      in_specs=[
          half, half, half, half,
          pl.BlockSpec((R, 1), lambda p, j: (j, 0)),
          vec, vec, vec,
          pl.BlockSpec((R, D), lambda p, j: (j, 0)),
          pl.BlockSpec((1, 1, R), lambda p, j: (j, 0, 0)),
          pl.BlockSpec((D, F), lambda p, j: (0, 0)),
          pl.BlockSpec((1, F), lambda p, j: (0, 0)),
      ],
      out_specs=pl.BlockSpec((NUM_GRAPHS, F), lambda p, j: (0, 0)),
      out_shape=jax.ShapeDtypeStruct((NUM_GRAPHS, F), jnp.float32),
      scratch_shapes=[pltpu.VMEM((8, D), jnp.float32),
                      pltpu.VMEM((NUM_GRAPHS, F), jnp.float32),
                      pltpu.VMEM((NUM_GRAPHS, 128), jnp.float32)],
  )(a_lo, a_hi, hs_lo, hs_hi, dinv, b2, g2, be2, x2, batch3d, fcW, fcb[None, :])


def kernel(x, edge_index, batch, W1, b1, g1, be1, W2, b2, g2, be2,
           W3, b3, g3, be3, fcW, fcb):
  src2d = edge_index[0].reshape(EROWS, K)
  dst2d = edge_index[1].reshape(EROWS, K)
  batch3d = batch.reshape(GRID, 1, R)
  zeros_full = jnp.zeros((N, 128), jnp.float32)
  ones_rows = jnp.ones((K, 128), jnp.float32)

  degp = _sc_deg(dst2d, ones_rows, zeros_full)

  hs1, dinv = _tc_first(x, W1, degp)
  p1 = _sc_agg_full(hs1, src2d, dst2d, zeros_full)
  hs_lo, hs_hi = _tc_layer([p1], [hs1], dinv, (b1, g1, be1), W2, 128, 256)

  a_lo, a_hi = _sc_agg_split(hs_lo, hs_hi, src2d, dst2d, zeros_full)
  hs_lo, hs_hi, x2 = _tc_layer([a_lo, a_hi], [hs_lo, hs_hi], dinv,
                               (b2, g2, be2), W3, 256, 256, want_x=True)

  a_lo, a_hi = _sc_agg_split(hs_lo, hs_hi, src2d, dst2d, zeros_full)
  return _tc_last(a_lo, a_hi, hs_lo, hs_hi, dinv, (b3, g3, be3), x2,
                  batch3d, fcW, fcb)


# speedup vs baseline: 19.6614x; 1.0008x over previous
import functools

import jax
import jax.numpy as jnp
from jax import lax
from jax.experimental import pallas as pl
from jax.experimental.pallas import tpu as pltpu
from jax.experimental.pallas import tpu_sc as plsc

N = 10000
E = 320000
NUM_GRAPHS = 64

NC = 2
NS = 16
NW = NC * NS
K = 125
EROWS = E // K
CH = 200
NCHUNKS = N // CH
KITER = -(-NCHUNKS // NS)
NH = 10240
R = 2000
GRID = N // R
EPS = 1e-5


def _mesh():
  return plsc.VectorSubcoreMesh(core_axis_name="c", subcore_axis_name="s")


def _rr_copy(src, dst, s):
  def go(k, _):
    idx = k * NS + s
    @pl.when(idx < NCHUNKS)
    def _():
      pltpu.sync_copy(src.at[pl.ds(idx * CH, CH)], dst.at[pl.ds(idx * CH, CH)])
    return 0
  lax.fori_loop(0, KITER, go, 0)


def _edge_pipeline(tbl, sh_agg, src_c, dst_c, bufs, gsems, ssems, nrows):
  gd = [None] * nrows
  sd = [None] * nrows
  gd[0] = pltpu.async_copy(tbl.at[src_c.at[0]], bufs[0], gsems[0])
  for j in range(1, nrows):
    p = j % 2
    if j >= 2:
      sd[j - 2].wait()
    gd[j] = pltpu.async_copy(tbl.at[src_c.at[j]], bufs[p], gsems[p])
    gd[j - 1].wait()
    q = (j - 1) % 2
    sd[j - 1] = pltpu.async_copy(bufs[q], sh_agg.at[dst_c.at[j - 1]],
                                 ssems[q], add=True)
  gd[nrows - 1].wait()
  q = (nrows - 1) % 2
  sd[nrows - 1] = pltpu.async_copy(bufs[q], sh_agg.at[dst_c.at[nrows - 1]],
                                   ssems[q], add=True)
  sd[nrows - 2].wait()
  sd[nrows - 1].wait()


def _sc_deg(dst2d, ones_rows, zeros_full):
  rows_per_worker = EROWS // NW
  FIRE = 16

  @functools.partial(
      pl.kernel,
      out_type=jax.ShapeDtypeStruct((NC, N, 128), jnp.float32),
      mesh=_mesh(),
      scratch_types=[
          pltpu.VMEM((rows_per_worker, K), jnp.int32),
          pltpu.VMEM((K, 128), jnp.float32),
          pltpu.VMEM_SHARED((N, 128), jnp.float32),
          pltpu.SemaphoreType.DMA,
      ],
  )
  def deg_kernel(dst_hbm, ones_hbm, zeros_hbm, out_hbm, dst_v, ones, sh_deg,
                 sem):
    c = lax.axis_index("c")
    s = lax.axis_index("s")
    w = c * NS + s
    pltpu.sync_copy(ones_hbm, ones)
    pltpu.sync_copy(dst_hbm.at[pl.ds(w * rows_per_worker, rows_per_worker)],
                    dst_v)
    _rr_copy(zeros_hbm, sh_deg, s)

    plsc.subcore_barrier()

    def grp(gi, _):
      ds = []
      for j in range(FIRE):
        ds.append(pltpu.async_copy(
            ones, sh_deg.at[dst_v.at[gi * FIRE + j]], sem, add=True))
      for d in ds:
        d.wait()
      return 0
    lax.fori_loop(0, rows_per_worker // FIRE, grp, 0)

    plsc.subcore_barrier()
    _rr_copy(sh_deg, out_hbm.at[c], s)

  return deg_kernel(dst2d, ones_rows, zeros_full)


def _sc_agg_split(hs_lo, hs_hi, src2d, dst2d, zeros_full):
  rows_per_sub = EROWS // NS
  IBS = 32
  n_chunks = rows_per_sub // IBS

  @functools.partial(
      pl.kernel,
      out_type=[jax.ShapeDtypeStruct((N, 128), jnp.float32),
                jax.ShapeDtypeStruct((N, 128), jnp.float32)],
      mesh=_mesh(),
      scratch_types=[
          pltpu.VMEM((IBS, K), jnp.int32),
          pltpu.VMEM((IBS, K), jnp.int32),
          pltpu.VMEM((K, 128), jnp.float32),
          pltpu.VMEM((K, 128), jnp.float32),
          pltpu.VMEM_SHARED((N, 128), jnp.float32),
          pltpu.SemaphoreType.DMA,
          pltpu.SemaphoreType.DMA,
          pltpu.SemaphoreType.DMA,
          pltpu.SemaphoreType.DMA,
      ],
  )
  def agg_kernel(lo_hbm, hi_hbm, src_hbm, dst_hbm, zeros_hbm, out_lo, out_hi,
                 src_c, dst_c, b0, b1, sh_agg, gs0, gs1, ss0, ss1):
    c = lax.axis_index("c")
    s = lax.axis_index("s")
    _rr_copy(zeros_hbm, sh_agg, s)
    plsc.subcore_barrier()

    def run(tbl, out):
      def chunk(ci, _):
        base = s * rows_per_sub + ci * IBS
        pltpu.sync_copy(src_hbm.at[pl.ds(base, IBS)], src_c)
        pltpu.sync_copy(dst_hbm.at[pl.ds(base, IBS)], dst_c)
        _edge_pipeline(tbl, sh_agg, src_c, dst_c, (b0, b1), (gs0, gs1),
                       (ss0, ss1), IBS)
        return 0
      lax.fori_loop(0, n_chunks, chunk, 0)

      plsc.subcore_barrier()
      _rr_copy(sh_agg, out, s)

    @pl.when(c == 0)
    def _():
      run(lo_hbm, out_lo)

    @pl.when(c == 1)
    def _():
      run(hi_hbm, out_hi)

  return agg_kernel(hs_lo, hs_hi, src2d, dst2d, zeros_full)


def _sc_agg_full(hs, src2d, dst2d, zeros_full):
  rows_per_worker = EROWS // NW
  IBS = 16
  n_chunks = rows_per_worker // IBS

  @functools.partial(
      pl.kernel,
      out_type=jax.ShapeDtypeStruct((NC, N, 128), jnp.float32),
      mesh=_mesh(),
      scratch_types=[
          pltpu.VMEM((IBS, K), jnp.int32),
          pltpu.VMEM((IBS, K), jnp.int32),
          pltpu.VMEM((K, 128), jnp.float32),
          pltpu.VMEM((K, 128), jnp.float32),
          pltpu.VMEM_SHARED((N, 128), jnp.float32),
          pltpu.SemaphoreType.DMA,
          pltpu.SemaphoreType.DMA,
          pltpu.SemaphoreType.DMA,
          pltpu.SemaphoreType.DMA,
      ],
  )
  def agg_kernel(hs_hbm, src_hbm, dst_hbm, zeros_hbm, out_hbm,
                 src_c, dst_c, b0, b1, sh_agg, gs0, gs1, ss0, ss1):
    c = lax.axis_index("c")
    s = lax.axis_index("s")
    w = c * NS + s
    _rr_copy(zeros_hbm, sh_agg, s)
    plsc.subcore_barrier()

    def chunk(ci, _):
      base = w * rows_per_worker + ci * IBS
      pltpu.sync_copy(src_hbm.at[pl.ds(base, IBS)], src_c)
      pltpu.sync_copy(dst_hbm.at[pl.ds(base, IBS)], dst_c)
      _edge_pipeline(hs_hbm, sh_agg, src_c, dst_c, (b0, b1), (gs0, gs1),
                     (ss0, ss1), IBS)
      return 0
    lax.fori_loop(0, n_chunks, chunk, 0)

    plsc.subcore_barrier()
    _rr_copy(sh_agg, out_hbm.at[c], s)

  return agg_kernel(hs, src2d, dst2d, zeros_full)


def _mm(a, b):
  return jnp.dot(a, b, preferred_element_type=jnp.float32,
                 precision=lax.Precision.HIGHEST)


def _tc_mm1(x, W1):
  def body(x_ref, w_ref, h_ref):
    h_ref[...] = _mm(x_ref[...], w_ref[...])

  return pl.pallas_call(
      body,
      grid=(GRID,),
      in_specs=[
          pl.BlockSpec((R, 128), lambda j: (j, 0)),
          pl.BlockSpec((128, 128), lambda j: (0, 0)),
      ],
      out_specs=pl.BlockSpec((R, 128), lambda j: (j, 0)),
      out_shape=jax.ShapeDtypeStruct((N, 128), jnp.float32),
  )(x, W1)


def _tc_first(h1raw, degp):
  def body(h_ref, degp_ref, hs_ref, dinv_ref):
    d = degp_ref[0, :, 0:1] + degp_ref[1, :, 0:1] + 1.0
    dinv = lax.rsqrt(d)
    dinv_ref[...] = dinv
    hs_ref[...] = h_ref[...] * dinv

  return pl.pallas_call(
      body,
      grid=(GRID,),
      in_specs=[
          pl.BlockSpec((R, 128), lambda j: (j, 0)),
          pl.BlockSpec((NC, R, 128), lambda j: (0, j, 0)),
      ],
      out_specs=[pl.BlockSpec((R, 128), lambda j: (j, 0)),
                 pl.BlockSpec((R, 1), lambda j: (j, 0))],
      out_shape=[jax.ShapeDtypeStruct((N, 128), jnp.float32),
                 jax.ShapeDtypeStruct((N, 1), jnp.float32)],
  )(h1raw, degp)


def _tc_layer(aggs, hss, dinv, bgbe, W, din, dout, want_x=False):
  H = dout // 2
  n_agg = len(aggs)
  n_hs = len(hss)

  def body(*refs):
    (agg_refs, hs_refs), rest = (refs[:n_agg], refs[n_agg:n_agg + n_hs]), \
        refs[n_agg + n_hs:]
    dinv_ref, b_ref, g_ref, be_ref, w_ref = rest[:5]
    outs = rest[5:-1]
    st_ref = rest[-1]
    p = pl.program_id(0)
    j = pl.program_id(1)

    if n_agg == 1:
      a3 = agg_refs[0][...]
      agg = a3[0] + a3[1]
    else:
      agg = jnp.concatenate([agg_refs[0][...], agg_refs[1][...]], axis=1)
    hs = hs_refs[0][...] if n_hs == 1 else \
        jnp.concatenate([hs_refs[0][...], hs_refs[1][...]], axis=1)
    z = (agg + hs) * dinv_ref[...] + b_ref[...]

    @pl.when(jnp.logical_and(p == 0, j == 0))
    def _():
      st_ref[...] = jnp.zeros_like(st_ref)

    @pl.when(p == 0)
    def _():
      st_ref[0:1, :] += jnp.sum(z, axis=0, keepdims=True)
      st_ref[1:2, :] += jnp.sum(z * z, axis=0, keepdims=True)

    @pl.when(p == 1)
    def _():
      @pl.when(j == 0)
      def _():
        mu = st_ref[0:1, :] / N
        var = st_ref[1:2, :] / N - mu * mu
        scale = g_ref[...] * lax.rsqrt(var + EPS)
        st_ref[2:3, :] = scale
        st_ref[3:4, :] = be_ref[...] - mu * scale

      xv = jnp.maximum(z * st_ref[2:3, :] + st_ref[3:4, :], 0.0)
      hs_next = _mm(xv, w_ref[...]) * dinv_ref[...]
      outs[0][...] = hs_next[:, :H]
      outs[1][...] = hs_next[:, H:]
      if want_x:
        outs[2][...] = xv

  in_specs = []
  ins = []
  for a in aggs:
    if a.ndim == 3:
      in_specs.append(pl.BlockSpec((NC, R, 128), lambda p, j: (0, j, 0)))
    else:
      in_specs.append(pl.BlockSpec((R, 128), lambda p, j: (j, 0)))
    ins.append(a)
  for h in hss:
    in_specs.append(pl.BlockSpec((R, h.shape[1]), lambda p, j: (j, 0)))
    ins.append(h)
  in_specs.append(pl.BlockSpec((R, 1), lambda p, j: (j, 0)))
  ins.append(dinv)
  for v in bgbe:
    in_specs.append(pl.BlockSpec((1, din), lambda p, j: (0, 0)))
    ins.append(v[None, :])
  in_specs.append(pl.BlockSpec((din, dout), lambda p, j: (0, 0)))
  ins.append(W)

  out_specs = [pl.BlockSpec((R, H), lambda p, j: (j, 0)),
               pl.BlockSpec((R, H), lambda p, j: (j, 0))]
  out_shape = [jax.ShapeDtypeStruct((N, H), jnp.float32)] * 2
  if want_x:
    out_specs.append(pl.BlockSpec((R, din), lambda p, j: (j, 0)))
    out_shape.append(jax.ShapeDtypeStruct((N, din), jnp.float32))

  return pl.pallas_call(
      body,
      grid=(2, GRID),
      in_specs=in_specs,
      out_specs=out_specs,
      out_shape=out_shape,
      scratch_shapes=[pltpu.VMEM((8, din), jnp.float32)],
  )(*ins)


def _tc_last(a_lo, a_hi, hs_lo, hs_hi, dinv, bgbe, x2, batch3d, fcW, fcb):
  D, F = 256, 256

  def body(alo_ref, ahi_ref, hlo_ref, hhi_ref, dinv_ref, b_ref, g_ref,
           be_ref, x2_ref, bt_ref, w_ref, fb_ref, out_ref, st_ref, acc_ref,
           cnt_ref):
    p = pl.program_id(0)
    j = pl.program_id(1)
    agg = jnp.concatenate([alo_ref[...], ahi_ref[...]], axis=1)
    hs = jnp.concatenate([hlo_ref[...], hhi_ref[...]], axis=1)
    z = (agg + hs) * dinv_ref[...] + b_ref[...]

    @pl.when(jnp.logical_and(p == 0, j == 0))
    def _():
      st_ref[...] = jnp.zeros_like(st_ref)
      acc_ref[...] = jnp.zeros_like(acc_ref)
      cnt_ref[...] = jnp.zeros_like(cnt_ref)

    @pl.when(p == 0)
    def _():
      st_ref[0:1, :] += jnp.sum(z, axis=0, keepdims=True)
      st_ref[1:2, :] += jnp.sum(z * z, axis=0, keepdims=True)

    @pl.when(p == 1)
    def _():
      @pl.when(j == 0)
      def _():
        mu = st_ref[0:1, :] / N
        var = st_ref[1:2, :] / N - mu * mu
        scale = g_ref[...] * lax.rsqrt(var + EPS)
        st_ref[2:3, :] = scale
        st_ref[3:4, :] = be_ref[...] - mu * scale

      x3 = jnp.maximum(z * st_ref[2:3, :] + st_ref[3:4, :], 0.0) + x2_ref[...]
      b = bt_ref[0, 0, :]
      gids = lax.broadcasted_iota(jnp.int32, (NUM_GRAPHS, R), 0)
      onehot = (b[None, :] == gids).astype(jnp.float32)
      acc_ref[...] += _mm(onehot, x3)
      csum = jnp.sum(onehot, axis=1, keepdims=True)
      cnt_ref[...] += jnp.broadcast_to(csum, (NUM_GRAPHS, 128))

      @pl.when(j == GRID - 1)
      def _():
        pooled = acc_ref[...] / jnp.maximum(cnt_ref[:, 0:1], 1.0)
        out_ref[...] = _mm(pooled, w_ref[...]) + fb_ref[...]

  half = pl.BlockSpec((R, 128), lambda p, j: (j, 0))
  vec = pl.BlockSpec((1, D), lambda p, j: (0, 0))
  b2, g2, be2 = (v[None, :] for v in bgbe)
  return pl.pallas_call(
      body,
      grid=(2, GRID),
      in_specs=[
          half, half, half, half,
          pl.BlockSpec((R, 1), lambda p, j: (j, 0)),
          vec, vec, vec,
          pl.BlockSpec((R, D), lambda p, j: (j, 0)),
          pl.BlockSpec((1, 1, R), lambda p, j: (j, 0, 0)),
          pl.BlockSpec((D, F), lambda p, j: (0, 0)),
          pl.BlockSpec((1, F), lambda p, j: (0, 0)),
      ],
      out_specs=pl.BlockSpec((NUM_GRAPHS, F), lambda p, j: (0, 0)),
      out_shape=jax.ShapeDtypeStruct((NUM_GRAPHS, F), jnp.float32),
      scratch_shapes=[pltpu.VMEM((8, D), jnp.float32),
                      pltpu.VMEM((NUM_GRAPHS, F), jnp.float32),
                      pltpu.VMEM((NUM_GRAPHS, 128), jnp.float32)],
  )(a_lo, a_hi, hs_lo, hs_hi, dinv, b2, g2, be2, x2, batch3d, fcW, fcb[None, :])


def kernel(x, edge_index, batch, W1, b1, g1, be1, W2, b2, g2, be2,
           W3, b3, g3, be3, fcW, fcb):
  src2d = edge_index[0].reshape(EROWS, K)
  dst2d = edge_index[1].reshape(EROWS, K)
  batch3d = batch.reshape(GRID, 1, R)
  zeros_full = jnp.zeros((N, 128), jnp.float32)
  ones_rows = jnp.ones((K, 128), jnp.float32)

  h1raw = _tc_mm1(x, W1)
  degp = _sc_deg(dst2d, ones_rows, zeros_full)

  hs1, dinv = _tc_first(h1raw, degp)
  p1 = _sc_agg_full(hs1, src2d, dst2d, zeros_full)
  hs_lo, hs_hi = _tc_layer([p1], [hs1], dinv, (b1, g1, be1), W2, 128, 256)

  a_lo, a_hi = _sc_agg_split(hs_lo, hs_hi, src2d, dst2d, zeros_full)
  hs_lo, hs_hi, x2 = _tc_layer([a_lo, a_hi], [hs_lo, hs_hi], dinv,
                               (b2, g2, be2), W3, 256, 256, want_x=True)

  a_lo, a_hi = _sc_agg_split(hs_lo, hs_hi, src2d, dst2d, zeros_full)
  return _tc_last(a_lo, a_hi, hs_lo, hs_hi, dinv, (b3, g3, be3), x2,
                  batch3d, fcW, fcb)
